# baseline (speedup 1.0000x reference)
import functools

import jax
import jax.numpy as jnp
from jax import lax
from jax.experimental import pallas as pl
from jax.experimental.pallas import tpu as pltpu
from jax.experimental.pallas import tpu_sc as plsc

B, S, D = 4, 8192, 2048
N = 1024
NC, NS = 2, 16
NW = NC * NS
ROWS_PER_W = (B * S) // NW
ENT_PER_W = (B * N) // NW
CH = 16
N_COPY = ROWS_PER_W // CH
N_SCAT = ENT_PER_W // CH
BIG = 2**30


_COPY_BLOCK = 1024


def _copy_idx_body(x_ref, vi_ref, o_ref, dst_ref, src_ref):
  o_ref[...] = x_ref[...]

  @pl.when(pl.program_id(0) == 0)
  def _():
    vi = vi_ref[...]
    rows = vi.shape[0]
    nxt = jnp.concatenate(
        [vi[:, 1:], jnp.full((rows, 1), -1, jnp.int32)], axis=1)
    is_last = vi != nxt
    j = lax.broadcasted_iota(jnp.int32, vi.shape, 1)
    w = jnp.where(is_last, j, BIG)
    k = 1
    while k < vi.shape[1]:
      shifted = jnp.concatenate(
          [w[:, k:], jnp.full((rows, k), BIG, jnp.int32)], axis=1)
      w = jnp.minimum(w, shifted)
      k *= 2
    b = lax.broadcasted_iota(jnp.int32, vi.shape, 0)
    dst_ref[...] = b * S + vi
    src_ref[...] = b * N + w


def _tc_copy_idx(text_flat, vision_indices):
  grid = (B * S) // _COPY_BLOCK
  out, dst, src = pl.pallas_call(
      _copy_idx_body,
      grid=(grid,),
      in_specs=[
          pl.BlockSpec((_COPY_BLOCK, D), lambda i: (i, 0)),
          pl.BlockSpec((B, N), lambda i: (0, 0)),
      ],
      out_specs=[
          pl.BlockSpec((_COPY_BLOCK, D), lambda i: (i, 0)),
          pl.BlockSpec((B, N), lambda i: (0, 0)),
          pl.BlockSpec((B, N), lambda i: (0, 0)),
      ],
      out_shape=[
          jax.ShapeDtypeStruct((B * S, D), jnp.float32),
          jax.ShapeDtypeStruct((B, N), jnp.int32),
          jax.ShapeDtypeStruct((B, N), jnp.int32),
      ],
  )(text_flat, vision_indices)
  return out, dst.reshape(-1), src.reshape(-1)


def _sc_body(text_hbm, img_hbm, dst_hbm, src_hbm, out_hbm,
             buf_a, buf_b, buf_c, dsti_m, srci_m, row0_v,
             sem_la, sem_lb, sem_lc, sem_sa, sem_sb, sem_sc3):
  c = lax.axis_index("c")
  s = lax.axis_index("s")
  wid = c * NS + s

  bufs = (buf_a, buf_b, buf_c)
  lsems = (sem_la, sem_lb, sem_lc)
  ssems = (sem_sa, sem_sb, sem_sc3)

  @pl.when(s < B // NC)
  def _():
    pltpu.sync_copy(text_hbm.at[pl.ds((c * (B // NC) + s) * S, 1)], row0_v)

  pltpu.sync_copy(dst_hbm.at[wid], dsti_m)
  pltpu.sync_copy(src_hbm.at[wid], srci_m)
  gath = [None] * N_SCAT
  scat = [None] * N_SCAT
  for t in range(2):
    gath[t] = pltpu.async_copy(
        img_hbm.at[srci_m.at[t]], bufs[t], lsems[t])
  for t in range(N_SCAT):
    if t + 2 < N_SCAT:
      if t >= 1:
        scat[t - 1].wait()
      gath[t + 2] = pltpu.async_copy(
          img_hbm.at[srci_m.at[t + 2]], bufs[(t + 2) % 3],
          lsems[(t + 2) % 3])
    gath[t].wait()
    scat[t] = pltpu.async_copy(
        bufs[t % 3], out_hbm.at[dsti_m.at[t]], ssems[t % 3])
  scat[N_SCAT - 3].wait()
  scat[N_SCAT - 2].wait()
  scat[N_SCAT - 1].wait()

  plsc.subcore_barrier()

  @pl.when(s < B // NC)
  def _():
    pltpu.sync_copy(row0_v, out_hbm.at[pl.ds((c * (B // NC) + s) * S, 1)])


@functools.partial(jax.jit, static_argnames=())
def kernel(image_embeddings, text_embeddings, vision_indices):
  text_flat = text_embeddings.reshape(B * S, D)
  img_flat = image_embeddings.reshape(B * N, D)
  out0, dst, src = _tc_copy_idx(text_flat, vision_indices)
  out_ref = jax.new_ref(out0)

  mesh = plsc.VectorSubcoreMesh(
      core_axis_name="c", subcore_axis_name="s",
      num_cores=NC, num_subcores=NS)
  sc = pl.kernel(
      _sc_body,
      out_type=(),
      mesh=mesh,
      scratch_types=[
          pltpu.VMEM((CH, D), jnp.float32),
          pltpu.VMEM((CH, D), jnp.float32),
          pltpu.VMEM((CH, D), jnp.float32),
          pltpu.VMEM((N_SCAT, CH), jnp.int32),
          pltpu.VMEM((N_SCAT, CH), jnp.int32),
          pltpu.VMEM((1, D), jnp.float32),
          pltpu.SemaphoreType.DMA,
          pltpu.SemaphoreType.DMA,
          pltpu.SemaphoreType.DMA,
          pltpu.SemaphoreType.DMA,
          pltpu.SemaphoreType.DMA,
          pltpu.SemaphoreType.DMA,
      ],
  )
  sc(text_flat, img_flat,
     dst.reshape(NW, N_SCAT, CH), src.reshape(NW, N_SCAT, CH), out_ref)
  return out_ref[...].reshape(B, S, D)


# speedup vs baseline: 1.0053x; 1.0053x over previous
import functools

import jax
import jax.numpy as jnp
from jax import lax
from jax.experimental import pallas as pl
from jax.experimental.pallas import tpu as pltpu
from jax.experimental.pallas import tpu_sc as plsc

B, S, D = 4, 8192, 2048
N = 1024
NC, NS = 2, 16
NW = NC * NS
ROWS_PER_W = (B * S) // NW
ENT_PER_W = (B * N) // NW
CH = 16
N_COPY = ROWS_PER_W // CH
N_SCAT = ENT_PER_W // CH
BIG = 2**30


_COPY_BLOCK = 1024


def _copy_idx_body(x_ref, vi_ref, o_ref, dst_ref, src_ref):
  o_ref[...] = x_ref[...]

  @pl.when(pl.program_id(0) == 0)
  def _():
    vi = vi_ref[...]
    rows = vi.shape[0]
    nxt = jnp.concatenate(
        [vi[:, 1:], jnp.full((rows, 1), -1, jnp.int32)], axis=1)
    is_last = vi != nxt
    j = lax.broadcasted_iota(jnp.int32, vi.shape, 1)
    w = jnp.where(is_last, j, BIG)
    k = 1
    while k < vi.shape[1]:
      shifted = jnp.concatenate(
          [w[:, k:], jnp.full((rows, k), BIG, jnp.int32)], axis=1)
      w = jnp.minimum(w, shifted)
      k *= 2
    b = lax.broadcasted_iota(jnp.int32, vi.shape, 0)
    dst_ref[...] = b * S + vi
    src_ref[...] = b * N + w


def _tc_copy_idx(text_flat, vision_indices):
  grid = (B * S) // _COPY_BLOCK
  out, dst, src = pl.pallas_call(
      _copy_idx_body,
      grid=(grid,),
      in_specs=[
          pl.BlockSpec((_COPY_BLOCK, D), lambda i: (i, 0)),
          pl.BlockSpec((B, N), lambda i: (0, 0)),
      ],
      out_specs=[
          pl.BlockSpec((_COPY_BLOCK, D), lambda i: (i, 0)),
          pl.BlockSpec((B, N), lambda i: (0, 0)),
          pl.BlockSpec((B, N), lambda i: (0, 0)),
      ],
      out_shape=[
          jax.ShapeDtypeStruct((B * S, D), jnp.float32),
          jax.ShapeDtypeStruct((B, N), jnp.int32),
          jax.ShapeDtypeStruct((B, N), jnp.int32),
      ],
  )(text_flat, vision_indices)
  return out, dst.reshape(-1), src.reshape(-1)


def _sc_body(text_hbm, img_hbm, dst_hbm, src_hbm, out_hbm,
             buf_a, buf_b, buf_c, dsti_a, dsti_b, dsti_c,
             srci_a, srci_b, srci_c, row0_v,
             sem_la, sem_lb, sem_lc, sem_sa, sem_sb, sem_sc3):
  c = lax.axis_index("c")
  s = lax.axis_index("s")
  wid = c * NS + s

  bufs = (buf_a, buf_b, buf_c)
  lsems = (sem_la, sem_lb, sem_lc)
  ssems = (sem_sa, sem_sb, sem_sc3)

  @pl.when(s < B // NC)
  def _():
    pltpu.sync_copy(text_hbm.at[pl.ds((c * (B // NC) + s) * S, 1)], row0_v)

  ebase = wid * ENT_PER_W
  dsts = (dsti_a, dsti_b, dsti_c)
  srcs = (srci_a, srci_b, srci_c)
  gath = [None] * N_SCAT
  scat = [None] * N_SCAT
  for t in range(2):
    off = ebase + t * CH
    pltpu.sync_copy(dst_hbm.at[pl.ds(off, CH)], dsts[t])
    pltpu.sync_copy(src_hbm.at[pl.ds(off, CH)], srcs[t])
    gath[t] = pltpu.async_copy(img_hbm.at[srcs[t]], bufs[t], lsems[t])
  for t in range(N_SCAT):
    if t + 2 < N_SCAT:
      if t >= 1:
        scat[t - 1].wait()
      off = ebase + (t + 2) * CH
      pltpu.sync_copy(dst_hbm.at[pl.ds(off, CH)], dsts[(t + 2) % 3])
      pltpu.sync_copy(src_hbm.at[pl.ds(off, CH)], srcs[(t + 2) % 3])
      gath[t + 2] = pltpu.async_copy(
          img_hbm.at[srcs[(t + 2) % 3]], bufs[(t + 2) % 3],
          lsems[(t + 2) % 3])
    gath[t].wait()
    scat[t] = pltpu.async_copy(
        bufs[t % 3], out_hbm.at[dsts[t % 3]], ssems[t % 3])
  scat[N_SCAT - 3].wait()
  scat[N_SCAT - 2].wait()
  scat[N_SCAT - 1].wait()

  plsc.subcore_barrier()

  @pl.when(s < B // NC)
  def _():
    pltpu.sync_copy(row0_v, out_hbm.at[pl.ds((c * (B // NC) + s) * S, 1)])


@functools.partial(jax.jit, static_argnames=())
def kernel(image_embeddings, text_embeddings, vision_indices):
  text_flat = text_embeddings.reshape(B * S, D)
  img_flat = image_embeddings.reshape(B * N, D)
  out0, dst, src = _tc_copy_idx(text_flat, vision_indices)
  out_ref = jax.new_ref(out0)

  mesh = plsc.VectorSubcoreMesh(
      core_axis_name="c", subcore_axis_name="s",
      num_cores=NC, num_subcores=NS)
  sc = pl.kernel(
      _sc_body,
      out_type=(),
      mesh=mesh,
      scratch_types=[
          pltpu.VMEM((CH, D), jnp.float32),
          pltpu.VMEM((CH, D), jnp.float32),
          pltpu.VMEM((CH, D), jnp.float32),
          pltpu.VMEM((CH,), jnp.int32),
          pltpu.VMEM((CH,), jnp.int32),
          pltpu.VMEM((CH,), jnp.int32),
          pltpu.VMEM((CH,), jnp.int32),
          pltpu.VMEM((CH,), jnp.int32),
          pltpu.VMEM((CH,), jnp.int32),
          pltpu.VMEM((1, D), jnp.float32),
          pltpu.SemaphoreType.DMA,
          pltpu.SemaphoreType.DMA,
          pltpu.SemaphoreType.DMA,
          pltpu.SemaphoreType.DMA,
          pltpu.SemaphoreType.DMA,
          pltpu.SemaphoreType.DMA,
      ],
  )
  sc(text_flat, img_flat, dst, src, out_ref)
  return out_ref[...].reshape(B, S, D)


# speedup vs baseline: 1.0103x; 1.0050x over previous
import functools

import jax
import jax.numpy as jnp
from jax import lax
from jax.experimental import pallas as pl
from jax.experimental.pallas import tpu as pltpu
from jax.experimental.pallas import tpu_sc as plsc

B, S, D = 4, 8192, 2048
N = 1024
NC, NS = 2, 16
NW = NC * NS
ROWS_PER_W = (B * S) // NW
ENT_PER_W = (B * N) // NW
CH = 16
N_COPY = ROWS_PER_W // CH
N_SCAT = ENT_PER_W // CH
BIG = 2**30


_COPY_BLOCK = 1024


def _copy_idx_body(x_ref, vi_ref, o_ref, dst_ref, src_ref):
  o_ref[...] = x_ref[...]

  @pl.when(pl.program_id(0) == 0)
  def _():
    vi = vi_ref[...]
    rows = vi.shape[0]
    nxt = jnp.concatenate(
        [vi[:, 1:], jnp.full((rows, 1), -1, jnp.int32)], axis=1)
    is_last = vi != nxt
    j = lax.broadcasted_iota(jnp.int32, vi.shape, 1)
    w = jnp.where(is_last, j, BIG)
    k = 1
    while k < vi.shape[1]:
      shifted = jnp.concatenate(
          [w[:, k:], jnp.full((rows, k), BIG, jnp.int32)], axis=1)
      w = jnp.minimum(w, shifted)
      k *= 2
    b = lax.broadcasted_iota(jnp.int32, vi.shape, 0)
    dst_ref[...] = b * S + vi
    src_ref[...] = b * N + w


def _tc_copy_idx(text_flat, vision_indices):
  grid = (B * S) // _COPY_BLOCK
  out, dst, src = pl.pallas_call(
      _copy_idx_body,
      grid=(grid,),
      in_specs=[
          pl.BlockSpec((_COPY_BLOCK, D), lambda i: (i, 0)),
          pl.BlockSpec((B, N), lambda i: (0, 0)),
      ],
      out_specs=[
          pl.BlockSpec((_COPY_BLOCK, D), lambda i: (i, 0)),
          pl.BlockSpec((B, N), lambda i: (0, 0)),
          pl.BlockSpec((B, N), lambda i: (0, 0)),
      ],
      out_shape=[
          jax.ShapeDtypeStruct((B * S, D), jnp.float32),
          jax.ShapeDtypeStruct((B, N), jnp.int32),
          jax.ShapeDtypeStruct((B, N), jnp.int32),
      ],
  )(text_flat, vision_indices)
  return out, dst, src


def _sc_body(text_hbm, img_hbm, dst_hbm, src_hbm, out_hbm,
             buf_a, buf_b, buf_c, dsti_a, dsti_b, dsti_c,
             srci_a, srci_b, srci_c, row0_v,
             sem_la, sem_lb, sem_lc, sem_sa, sem_sb, sem_sc3):
  c = lax.axis_index("c")
  s = lax.axis_index("s")
  wid = c * NS + s

  bufs = (buf_a, buf_b, buf_c)
  lsems = (sem_la, sem_lb, sem_lc)
  ssems = (sem_sa, sem_sb, sem_sc3)

  @pl.when(s < B // NC)
  def _():
    pltpu.sync_copy(text_hbm.at[pl.ds((c * (B // NC) + s) * S, 1)], row0_v)

  WPB = NW // B
  erow = wid // WPB
  ecol = (wid % WPB) * ENT_PER_W
  dsts = (dsti_a, dsti_b, dsti_c)
  srcs = (srci_a, srci_b, srci_c)
  gath = [None] * N_SCAT
  scat = [None] * N_SCAT
  for t in range(2):
    off = ecol + t * CH
    pltpu.sync_copy(dst_hbm.at[erow, pl.ds(off, CH)], dsts[t])
    pltpu.sync_copy(src_hbm.at[erow, pl.ds(off, CH)], srcs[t])
    gath[t] = pltpu.async_copy(img_hbm.at[srcs[t]], bufs[t], lsems[t])
  for t in range(N_SCAT):
    if t + 2 < N_SCAT:
      if t >= 1:
        scat[t - 1].wait()
      off = ecol + (t + 2) * CH
      pltpu.sync_copy(dst_hbm.at[erow, pl.ds(off, CH)], dsts[(t + 2) % 3])
      pltpu.sync_copy(src_hbm.at[erow, pl.ds(off, CH)], srcs[(t + 2) % 3])
      gath[t + 2] = pltpu.async_copy(
          img_hbm.at[srcs[(t + 2) % 3]], bufs[(t + 2) % 3],
          lsems[(t + 2) % 3])
    gath[t].wait()
    scat[t] = pltpu.async_copy(
        bufs[t % 3], out_hbm.at[dsts[t % 3]], ssems[t % 3])
  scat[N_SCAT - 3].wait()
  scat[N_SCAT - 2].wait()
  scat[N_SCAT - 1].wait()

  plsc.subcore_barrier()

  @pl.when(s < B // NC)
  def _():
    pltpu.sync_copy(row0_v, out_hbm.at[pl.ds((c * (B // NC) + s) * S, 1)])


@functools.partial(jax.jit, static_argnames=())
def kernel(image_embeddings, text_embeddings, vision_indices):
  text_flat = text_embeddings.reshape(B * S, D)
  img_flat = image_embeddings.reshape(B * N, D)
  out0, dst, src = _tc_copy_idx(text_flat, vision_indices)
  out_ref = jax.new_ref(out0)

  mesh = plsc.VectorSubcoreMesh(
      core_axis_name="c", subcore_axis_name="s",
      num_cores=NC, num_subcores=NS)
  sc = pl.kernel(
      _sc_body,
      out_type=(),
      mesh=mesh,
      scratch_types=[
          pltpu.VMEM((CH, D), jnp.float32),
          pltpu.VMEM((CH, D), jnp.float32),
          pltpu.VMEM((CH, D), jnp.float32),
          pltpu.VMEM((CH,), jnp.int32),
          pltpu.VMEM((CH,), jnp.int32),
          pltpu.VMEM((CH,), jnp.int32),
          pltpu.VMEM((CH,), jnp.int32),
          pltpu.VMEM((CH,), jnp.int32),
          pltpu.VMEM((CH,), jnp.int32),
          pltpu.VMEM((1, D), jnp.float32),
          pltpu.SemaphoreType.DMA,
          pltpu.SemaphoreType.DMA,
          pltpu.SemaphoreType.DMA,
          pltpu.SemaphoreType.DMA,
          pltpu.SemaphoreType.DMA,
          pltpu.SemaphoreType.DMA,
      ],
  )
  sc(text_flat, img_flat, dst, src, out_ref)
  return out_ref[...].reshape(B, S, D)
